# Initial kernel scaffold; baseline (speedup 1.0000x reference)
import functools

import jax
import jax.numpy as jnp
from jax import lax
from jax.experimental import pallas as pl
from jax.experimental.pallas import tpu as pltpu
from jax.experimental.pallas import tpu_sc as plsc

N = 10000
E = 320000
D = 128

NC = 2
NS = 16
NW = NC * NS

EPT = E // NW
CH = 80
NCHUNK = EPT // CH
RPT = N // NS
RB = 125
NRB = RPT // RB

_mesh = plsc.VectorSubcoreMesh(
    core_axis_name="c", subcore_axis_name="s", num_cores=NC, num_subcores=NS
)


@functools.partial(
    pl.kernel,
    out_type=(
        jax.ShapeDtypeStruct((N, 16), jnp.float32),
        jax.ShapeDtypeStruct((N, 16), jnp.float32),
    ),
    mesh=_mesh,
    scratch_types=(
        pltpu.VMEM_SHARED((N, 16), jnp.float32),
        pltpu.VMEM((CH,), jnp.int32),
        pltpu.VMEM((CH, 16), jnp.float32),
        pltpu.VMEM((RPT, 16), jnp.float32),
        pltpu.SemaphoreType.DMA,
    ),
)
def _deg_kernel(dst_hbm, out0_hbm, out1_hbm, acc, dst_v, ones_v, buf_v, sem):
    c = lax.axis_index("c")
    s = lax.axis_index("s")
    wid = c * NS + s

    def fill(i, _):
        ones_v[i, :] = jnp.ones((16,), jnp.float32)
        return 0

    lax.fori_loop(0, CH, fill, 0)

    def fillz(i, _):
        buf_v[i, :] = jnp.zeros((16,), jnp.float32)
        return 0

    lax.fori_loop(0, RPT, fillz, 0)
    pltpu.sync_copy(buf_v, acc.at[pl.ds(s * RPT, RPT)])
    plsc.subcore_barrier()

    ebase = wid * EPT

    def chunk(i, _):
        pltpu.sync_copy(dst_hbm.at[pl.ds(ebase + i * CH, CH)], dst_v)
        pltpu.sync_copy(ones_v, acc.at[dst_v], add=True)
        return 0

    lax.fori_loop(0, NCHUNK, chunk, 0)
    plsc.subcore_barrier()

    pltpu.sync_copy(acc.at[pl.ds(s * RPT, RPT)], buf_v)

    @pl.when(c == 0)
    def _():
        pltpu.sync_copy(buf_v, out0_hbm.at[pl.ds(s * RPT, RPT)])

    @pl.when(c == 1)
    def _():
        pltpu.sync_copy(buf_v, out1_hbm.at[pl.ds(s * RPT, RPT)])


@functools.partial(
    pl.kernel,
    out_type=(
        jax.ShapeDtypeStruct((N, D), jnp.float32),
        jax.ShapeDtypeStruct((N, D), jnp.float32),
    ),
    mesh=_mesh,
    scratch_types=(
        pltpu.VMEM_SHARED((N, D), jnp.float32),
        pltpu.VMEM((CH,), jnp.int32),
        pltpu.VMEM((CH,), jnp.int32),
        pltpu.VMEM((CH, D), jnp.float32),
        pltpu.VMEM((RB, D), jnp.float32),
        pltpu.SemaphoreType.DMA,
    ),
)
def _agg_kernel(h_hbm, src_hbm, dst_hbm, out0_hbm, out1_hbm,
                acc, src_v, dst_v, rows_v, buf_v, sem):
    c = lax.axis_index("c")
    s = lax.axis_index("s")
    wid = c * NS + s

    def fillz(i, _):
        for j in range(D // 16):
            buf_v[i, pl.ds(j * 16, 16)] = jnp.zeros((16,), jnp.float32)
        return 0

    lax.fori_loop(0, RB, fillz, 0)
    for t in range(NRB):
        pltpu.sync_copy(buf_v, acc.at[pl.ds(s * RPT + t * RB, RB)])
    plsc.subcore_barrier()

    ebase = wid * EPT

    def chunk(i, _):
        base = ebase + i * CH
        pltpu.sync_copy(src_hbm.at[pl.ds(base, CH)], src_v)
        pltpu.sync_copy(dst_hbm.at[pl.ds(base, CH)], dst_v)
        pltpu.async_copy(h_hbm.at[src_v], rows_v, sem).wait()
        pltpu.sync_copy(rows_v, acc.at[dst_v], add=True)
        return 0

    lax.fori_loop(0, NCHUNK, chunk, 0)
    plsc.subcore_barrier()

    for t in range(NRB):
        r0 = s * RPT + t * RB
        pltpu.sync_copy(acc.at[pl.ds(r0, RB)], buf_v)

        @pl.when(c == 0)
        def _():
            pltpu.sync_copy(buf_v, out0_hbm.at[pl.ds(r0, RB)])

        @pl.when(c == 1)
        def _():
            pltpu.sync_copy(buf_v, out1_hbm.at[pl.ds(r0, RB)])


RBLK = 2000
GRID = N // RBLK


def _dis_block(d0_ref, d1_ref):
    deg = d0_ref[:, 0:1] + d1_ref[:, 0:1] + 1.0
    return lax.rsqrt(deg)


def _mm_scale_body(x_ref, w_ref, d0_ref, d1_ref, o_ref):
    dis = _dis_block(d0_ref, d1_ref)
    h = jnp.dot(x_ref[...], w_ref[...], preferred_element_type=jnp.float32,
                precision=lax.Precision.HIGHEST)
    o_ref[...] = h * dis


def _mid_body(a0_ref, a1_ref, hp_ref, d0_ref, d1_ref, b_ref, w_ref, o_ref):
    dis = _dis_block(d0_ref, d1_ref)
    z = (a0_ref[...] + a1_ref[...] + hp_ref[...]) * dis + b_ref[...]
    z = jnp.maximum(z, 0.0)
    h = jnp.dot(z, w_ref[...], preferred_element_type=jnp.float32,
                precision=lax.Precision.HIGHEST)
    o_ref[...] = h * dis


def _final_body(a0_ref, a1_ref, hp_ref, d0_ref, d1_ref, b_ref, o_ref):
    dis = _dis_block(d0_ref, d1_ref)
    o_ref[...] = (a0_ref[...] + a1_ref[...] + hp_ref[...]) * dis + b_ref[...]


_row_spec = pl.BlockSpec((RBLK, D), lambda i: (i, 0))
_deg_spec = pl.BlockSpec((RBLK, 16), lambda i: (i, 0))
_w_spec = pl.BlockSpec((D, D), lambda i: (0, 0))
_b_spec = pl.BlockSpec((1, D), lambda i: (0, 0))
_out_shape = jax.ShapeDtypeStruct((N, D), jnp.float32)


_mm_scale = pl.pallas_call(
    _mm_scale_body,
    grid=(GRID,),
    in_specs=[_row_spec, _w_spec, _deg_spec, _deg_spec],
    out_specs=_row_spec,
    out_shape=_out_shape,
)

_mid = pl.pallas_call(
    _mid_body,
    grid=(GRID,),
    in_specs=[_row_spec, _row_spec, _row_spec, _deg_spec, _deg_spec,
              _b_spec, _w_spec],
    out_specs=_row_spec,
    out_shape=_out_shape,
)

_final = pl.pallas_call(
    _final_body,
    grid=(GRID,),
    in_specs=[_row_spec, _row_spec, _row_spec, _deg_spec, _deg_spec, _b_spec],
    out_specs=_row_spec,
    out_shape=_out_shape,
)


def kernel(x, edge_index, W1, b1, W2, b2):
    src = edge_index[0].astype(jnp.int32)
    dst = edge_index[1].astype(jnp.int32)
    b1r = b1.reshape(1, D)
    b2r = b2.reshape(1, D)

    deg0, deg1 = _deg_kernel(dst)
    h1p = _mm_scale(x, W1, deg0, deg1)
    a10, a11 = _agg_kernel(h1p, src, dst)
    h2p = _mid(a10, a11, h1p, deg0, deg1, b1r, W2)
    a20, a21 = _agg_kernel(h2p, src, dst)
    return _final(a20, a21, h2p, deg0, deg1, b2r)


# speedup vs baseline: 18.9652x; 18.9652x over previous
import functools

import jax
import jax.numpy as jnp
from jax import lax
from jax.experimental import pallas as pl
from jax.experimental.pallas import tpu as pltpu
from jax.experimental.pallas import tpu_sc as plsc

N = 10000
NP = 10240
E = 320000
D = 128

NC = 2
NS = 16
NW = NC * NS

EPT = E // NW
CH = 80
NCHUNK = EPT // CH
CHD = 40
NCHUNKD = EPT // CHD
RPT = NP // NS
RB = CH
NRB = RPT // RB
RBD = 32
NRBD = RPT // RBD

_mesh = plsc.VectorSubcoreMesh(
    core_axis_name="c", subcore_axis_name="s", num_cores=NC, num_subcores=NS
)


@functools.partial(
    pl.kernel,
    out_type=jax.ShapeDtypeStruct((NC, NP, D), jnp.float32),
    mesh=_mesh,
    scratch_types=(
        pltpu.VMEM_SHARED((NP, D), jnp.float32),
        pltpu.VMEM((2, CH), jnp.int32),
        pltpu.VMEM((2, CH), jnp.int32),
        pltpu.VMEM((CH,), jnp.int32),
        pltpu.VMEM((CH,), jnp.int32),
        pltpu.VMEM((CH, D), jnp.float32),
        pltpu.VMEM((CH, D), jnp.float32),
        pltpu.SemaphoreType.DMA,
        pltpu.SemaphoreType.DMA,
        pltpu.SemaphoreType.DMA,
        pltpu.SemaphoreType.DMA,
    ),
)
def _agg_kernel(h_hbm, sd_hbm, out_hbm,
                acc, sd0, sd1, dv0, dv1, rows0, rows1,
                gsem0, gsem1, isem0, isem1):
    c = lax.axis_index("c")
    s = lax.axis_index("s")
    wid = c * NS + s

    def fillz(i, _):
        for j in range(D // 16):
            rows0[i, pl.ds(j * 16, 16)] = jnp.zeros((16,), jnp.float32)
        return 0

    lax.fori_loop(0, RB, fillz, 0)
    for t in range(NRB):
        pltpu.sync_copy(rows0, acc.at[pl.ds(s * RPT + t * RB, RB)])
    plsc.subcore_barrier()

    def iload(j, sd, isem):
        pltpu.async_copy(sd_hbm.at[wid, j], sd, isem)

    def iwait(j, sd, isem):
        pltpu.make_async_copy(sd_hbm.at[wid, j], sd, isem).wait()

    def dcopy(sd, dv):
        for g in range(CH // 16):
            dv[pl.ds(g * 16, 16)] = sd[1, pl.ds(g * 16, 16)]

    def gather(j, sd, buf, gsem):
        pltpu.async_copy(h_hbm.at[sd.at[0]], buf, gsem)

    def gwait(j, sd, buf, gsem):
        pltpu.make_async_copy(h_hbm.at[sd.at[0]], buf, gsem).wait()

    def scat(buf, dv):
        pltpu.sync_copy(buf, acc.at[dv], add=True)

    iload(0, sd0, isem0)
    iwait(0, sd0, isem0)
    dcopy(sd0, dv0)
    gather(0, sd0, rows0, gsem0)
    iload(1, sd1, isem1)

    def pair(k, _):
        i = 2 * k + 1
        iwait(i, sd1, isem1)
        dcopy(sd1, dv1)
        gather(i, sd1, rows1, gsem1)
        gwait(i - 1, sd0, rows0, gsem0)
        scat(rows0, dv0)
        iload(i + 1, sd0, isem0)
        iwait(i + 1, sd0, isem0)
        dcopy(sd0, dv0)
        gather(i + 1, sd0, rows0, gsem0)
        gwait(i, sd1, rows1, gsem1)
        scat(rows1, dv1)

        @pl.when(i + 2 < NCHUNK)
        def _():
            iload(i + 2, sd1, isem1)

        return 0

    lax.fori_loop(0, (NCHUNK - 1) // 2, pair, 0)
    gwait(NCHUNK - 1, sd0, rows0, gsem0)
    scat(rows0, dv0)
    plsc.subcore_barrier()

    for t in range(NRB):
        r0 = s * RPT + t * RB
        pltpu.sync_copy(acc.at[pl.ds(r0, RB)], rows0)
        pltpu.sync_copy(rows0, out_hbm.at[c, pl.ds(r0, RB)])


RBLK = 2000
GRID = N // RBLK


def _dis_body(dega_ref, o_ref):
    deg = dega_ref[0, :, 0:1] + dega_ref[1, :, 0:1] + 1.0
    o_ref[...] = jnp.broadcast_to(lax.rsqrt(deg), (RBLK, 16))


def _mm_scale_body(x_ref, w_ref, dis_ref, o_ref):
    dis = dis_ref[:, 0:1]
    h = jnp.dot(x_ref[...], w_ref[...], preferred_element_type=jnp.float32,
                precision=lax.Precision.HIGHEST)
    o_ref[...] = h * dis


def _mid_body(a_ref, hp_ref, dis_ref, b_ref, w_ref, o_ref):
    dis = dis_ref[:, 0:1]
    z = (a_ref[0] + a_ref[1] + hp_ref[...]) * dis + b_ref[...]
    z = jnp.maximum(z, 0.0)
    h = jnp.dot(z, w_ref[...], preferred_element_type=jnp.float32,
                precision=lax.Precision.HIGHEST)
    o_ref[...] = h * dis


def _final_body(a_ref, hp_ref, dis_ref, b_ref, o_ref):
    dis = dis_ref[:, 0:1]
    o_ref[...] = (a_ref[0] + a_ref[1] + hp_ref[...]) * dis + b_ref[...]


_row_spec = pl.BlockSpec((RBLK, D), lambda i: (i, 0))
_agg_spec = pl.BlockSpec((NC, RBLK, D), lambda i: (0, i, 0))
_dis_spec = pl.BlockSpec((RBLK, 16), lambda i: (i, 0))
_w_spec = pl.BlockSpec((D, D), lambda i: (0, 0))
_b_spec = pl.BlockSpec((1, D), lambda i: (0, 0))
_out_shape = jax.ShapeDtypeStruct((N, D), jnp.float32)

_dis_tc = pl.pallas_call(
    _dis_body,
    grid=(GRID,),
    in_specs=[pl.BlockSpec((NC, RBLK, D), lambda i: (0, i, 0))],
    out_specs=pl.BlockSpec((RBLK, 16), lambda i: (i, 0)),
    out_shape=jax.ShapeDtypeStruct((N, 16), jnp.float32),
)

_mm_scale = pl.pallas_call(
    _mm_scale_body,
    grid=(GRID,),
    in_specs=[_row_spec, _w_spec, _dis_spec],
    out_specs=_row_spec,
    out_shape=_out_shape,
)

_mid = pl.pallas_call(
    _mid_body,
    grid=(GRID,),
    in_specs=[_agg_spec, _row_spec, _dis_spec, _b_spec, _w_spec],
    out_specs=_row_spec,
    out_shape=_out_shape,
)

_final = pl.pallas_call(
    _final_body,
    grid=(GRID,),
    in_specs=[_agg_spec, _row_spec, _dis_spec, _b_spec],
    out_specs=_row_spec,
    out_shape=_out_shape,
)


def kernel(x, edge_index, W1, b1, W2, b2):
    src = edge_index[0].astype(jnp.int32)
    dst = edge_index[1].astype(jnp.int32)
    b1r = b1.reshape(1, D)
    b2r = b2.reshape(1, D)

    sd = jnp.stack(
        [src.reshape(NW, NCHUNK, CH), dst.reshape(NW, NCHUNK, CH)], axis=2
    )

    ones_t = jnp.ones((N, D), jnp.float32)
    dega = _agg_kernel(ones_t, sd)
    dis16 = _dis_tc(dega)
    h1p = _mm_scale(x, W1, dis16)
    a1 = _agg_kernel(h1p, sd)
    h2p = _mid(a1, h1p, dis16, b1r, W2)
    a2 = _agg_kernel(h2p, sd)
    return _final(a2, h2p, dis16, b2r)


# speedup vs baseline: 22.2534x; 1.1734x over previous
import functools

import jax
import jax.numpy as jnp
from jax import lax
from jax.experimental import pallas as pl
from jax.experimental.pallas import tpu as pltpu
from jax.experimental.pallas import tpu_sc as plsc

N = 10000
NP = 10240
E = 320000
D = 128

NC = 2
NS = 16
NW = NC * NS

EPT = E // NW
CH = 128
NCHUNK = 81
EPT_P = NCHUNK * CH
RPT = NP // NS
RB = CH
NRB = RPT // RB

_mesh = plsc.VectorSubcoreMesh(
    core_axis_name="c", subcore_axis_name="s", num_cores=NC, num_subcores=NS
)


@functools.partial(
    pl.kernel,
    out_type=jax.ShapeDtypeStruct((NC, NP, D), jnp.float32),
    mesh=_mesh,
    scratch_types=(
        pltpu.VMEM_SHARED((NP, D), jnp.float32),
        pltpu.VMEM((2, CH), jnp.int32),
        pltpu.VMEM((2, CH), jnp.int32),
        pltpu.VMEM((CH,), jnp.int32),
        pltpu.VMEM((CH,), jnp.int32),
        pltpu.VMEM((CH, D), jnp.float32),
        pltpu.VMEM((CH, D), jnp.float32),
        pltpu.SemaphoreType.DMA,
        pltpu.SemaphoreType.DMA,
        pltpu.SemaphoreType.DMA,
        pltpu.SemaphoreType.DMA,
    ),
)
def _agg_kernel(h_hbm, sd_hbm, out_hbm,
                acc, sd0, sd1, dv0, dv1, rows0, rows1,
                gsem0, gsem1, isem0, isem1):
    c = lax.axis_index("c")
    s = lax.axis_index("s")
    wid = c * NS + s

    def fillz(i, _):
        for j in range(D // 16):
            rows0[i, pl.ds(j * 16, 16)] = jnp.zeros((16,), jnp.float32)
        return 0

    lax.fori_loop(0, RB, fillz, 0)
    for t in range(NRB):
        pltpu.sync_copy(rows0, acc.at[pl.ds(s * RPT + t * RB, RB)])
    plsc.subcore_barrier()

    def iload(j, sd, isem):
        pltpu.async_copy(sd_hbm.at[wid, j], sd, isem)

    def iwait(j, sd, isem):
        pltpu.make_async_copy(sd_hbm.at[wid, j], sd, isem).wait()

    def dcopy(sd, dv):
        for g in range(CH // 16):
            dv[pl.ds(g * 16, 16)] = sd[1, pl.ds(g * 16, 16)]

    def gather(j, sd, buf, gsem):
        pltpu.async_copy(h_hbm.at[sd.at[0]], buf, gsem)

    def gwait(j, sd, buf, gsem):
        pltpu.make_async_copy(h_hbm.at[sd.at[0]], buf, gsem).wait()

    def scat(buf, dv):
        pltpu.sync_copy(buf, acc.at[dv], add=True)

    iload(0, sd0, isem0)
    iwait(0, sd0, isem0)
    dcopy(sd0, dv0)
    gather(0, sd0, rows0, gsem0)
    iload(1, sd1, isem1)

    def pair(k, _):
        i = 2 * k + 1
        iwait(i, sd1, isem1)
        dcopy(sd1, dv1)
        gather(i, sd1, rows1, gsem1)
        gwait(i - 1, sd0, rows0, gsem0)
        scat(rows0, dv0)
        iload(i + 1, sd0, isem0)
        iwait(i + 1, sd0, isem0)
        dcopy(sd0, dv0)
        gather(i + 1, sd0, rows0, gsem0)
        gwait(i, sd1, rows1, gsem1)
        scat(rows1, dv1)

        @pl.when(i + 2 < NCHUNK)
        def _():
            iload(i + 2, sd1, isem1)

        return 0

    lax.fori_loop(0, (NCHUNK - 1) // 2, pair, 0)
    gwait(NCHUNK - 1, sd0, rows0, gsem0)
    scat(rows0, dv0)
    plsc.subcore_barrier()

    for t in range(NRB):
        r0 = s * RPT + t * RB
        pltpu.sync_copy(acc.at[pl.ds(r0, RB)], rows0)
        pltpu.sync_copy(rows0, out_hbm.at[c, pl.ds(r0, RB)])


RBLK = 2000
GRID = N // RBLK


def _dis_body(dega_ref, o_ref):
    deg = dega_ref[0, :, 0:1] + dega_ref[1, :, 0:1] + 1.0
    o_ref[...] = jnp.broadcast_to(lax.rsqrt(deg), (RBLK, 16))


def _mm_scale_body(x_ref, w_ref, dis_ref, o_ref):
    dis = dis_ref[:, 0:1]
    h = jnp.dot(x_ref[...], w_ref[...], preferred_element_type=jnp.float32,
                precision=lax.Precision.HIGHEST)
    o_ref[...] = h * dis


def _mid_body(a_ref, hp_ref, dis_ref, b_ref, w_ref, o_ref):
    dis = dis_ref[:, 0:1]
    z = (a_ref[0] + a_ref[1] + hp_ref[...]) * dis + b_ref[...]
    z = jnp.maximum(z, 0.0)
    h = jnp.dot(z, w_ref[...], preferred_element_type=jnp.float32,
                precision=lax.Precision.HIGHEST)
    o_ref[...] = h * dis


def _final_body(a_ref, hp_ref, dis_ref, b_ref, o_ref):
    dis = dis_ref[:, 0:1]
    o_ref[...] = (a_ref[0] + a_ref[1] + hp_ref[...]) * dis + b_ref[...]


_row_spec = pl.BlockSpec((RBLK, D), lambda i: (i, 0))
_agg_spec = pl.BlockSpec((NC, RBLK, D), lambda i: (0, i, 0))
_dis_spec = pl.BlockSpec((RBLK, 16), lambda i: (i, 0))
_w_spec = pl.BlockSpec((D, D), lambda i: (0, 0))
_b_spec = pl.BlockSpec((1, D), lambda i: (0, 0))
_out_shape = jax.ShapeDtypeStruct((N, D), jnp.float32)

_dis_tc = pl.pallas_call(
    _dis_body,
    grid=(GRID,),
    in_specs=[pl.BlockSpec((NC, RBLK, D), lambda i: (0, i, 0))],
    out_specs=pl.BlockSpec((RBLK, 16), lambda i: (i, 0)),
    out_shape=jax.ShapeDtypeStruct((N, 16), jnp.float32),
)

_mm_scale = pl.pallas_call(
    _mm_scale_body,
    grid=(GRID,),
    in_specs=[_row_spec, _w_spec, _dis_spec],
    out_specs=_row_spec,
    out_shape=_out_shape,
)

_mid = pl.pallas_call(
    _mid_body,
    grid=(GRID,),
    in_specs=[_agg_spec, _row_spec, _dis_spec, _b_spec, _w_spec],
    out_specs=_row_spec,
    out_shape=_out_shape,
)

_final = pl.pallas_call(
    _final_body,
    grid=(GRID,),
    in_specs=[_agg_spec, _row_spec, _dis_spec, _b_spec],
    out_specs=_row_spec,
    out_shape=_out_shape,
)


def kernel(x, edge_index, W1, b1, W2, b2):
    src = edge_index[0].astype(jnp.int32)
    dst = edge_index[1].astype(jnp.int32)
    b1r = b1.reshape(1, D)
    b2r = b2.reshape(1, D)

    padlen = EPT_P - EPT
    w_ids = jnp.arange(NW, dtype=jnp.int32)[:, None]
    p_ids = jnp.arange(padlen, dtype=jnp.int32)[None, :]
    pad_src = (w_ids * 131 + p_ids * 997) % N
    pad_dst = N + (w_ids * 7 + p_ids) % (NP - N)
    srcp = jnp.concatenate([src.reshape(NW, EPT), pad_src], axis=1)
    dstp = jnp.concatenate([dst.reshape(NW, EPT), pad_dst], axis=1)
    sd = jnp.stack(
        [srcp.reshape(NW, NCHUNK, CH), dstp.reshape(NW, NCHUNK, CH)], axis=2
    )

    ones_t = jnp.ones((N, D), jnp.float32)
    dega = _agg_kernel(ones_t, sd)
    dis16 = _dis_tc(dega)
    h1p = _mm_scale(x, W1, dis16)
    a1 = _agg_kernel(h1p, sd)
    h2p = _mid(a1, h1p, dis16, b1r, W2)
    a2 = _agg_kernel(h2p, sd)
    return _final(a2, h2p, dis16, b2r)


# speedup vs baseline: 22.5670x; 1.0141x over previous
import functools

import jax
import jax.numpy as jnp
from jax import lax
from jax.experimental import pallas as pl
from jax.experimental.pallas import tpu as pltpu
from jax.experimental.pallas import tpu_sc as plsc

N = 10000
NP = 10240
E = 320000
D = 128

NC = 2
NS = 16
NW = NC * NS

EPT = E // NW
CH = 128
NCHUNK = 81
EPT_P = NCHUNK * CH
RPT = NP // NS
RB = CH
NRB = RPT // RB

_mesh = plsc.VectorSubcoreMesh(
    core_axis_name="c", subcore_axis_name="s", num_cores=NC, num_subcores=NS
)


@functools.partial(
    pl.kernel,
    out_type=jax.ShapeDtypeStruct((NC, NP, D), jnp.float32),
    mesh=_mesh,
    scratch_types=(
        pltpu.VMEM_SHARED((NP, D), jnp.float32),
        pltpu.VMEM((2, CH), jnp.int32),
        pltpu.VMEM((2, CH), jnp.int32),
        pltpu.VMEM((CH,), jnp.int32),
        pltpu.VMEM((CH,), jnp.int32),
        pltpu.VMEM((CH, D), jnp.float32),
        pltpu.VMEM((CH, D), jnp.float32),
        pltpu.SemaphoreType.DMA,
        pltpu.SemaphoreType.DMA,
        pltpu.SemaphoreType.DMA,
        pltpu.SemaphoreType.DMA,
    ),
)
def _agg_kernel(h_hbm, sd_hbm, out_hbm,
                acc, sd0, sd1, dv0, dv1, rows0, rows1,
                gsem0, gsem1, isem0, isem1):
    c = lax.axis_index("c")
    s = lax.axis_index("s")
    wid = c * NS + s

    def fillz(i, _):
        for j in range(D // 16):
            rows0[i, pl.ds(j * 16, 16)] = jnp.zeros((16,), jnp.float32)
        return 0

    lax.fori_loop(0, RB, fillz, 0)
    for t in range(NRB):
        pltpu.sync_copy(rows0, acc.at[pl.ds(s * RPT + t * RB, RB)])
    plsc.subcore_barrier()

    def iload(j, sd, isem):
        pltpu.async_copy(sd_hbm.at[wid, j], sd, isem)

    def iwait(j, sd, isem):
        pltpu.make_async_copy(sd_hbm.at[wid, j], sd, isem).wait()

    def dcopy(sd, dv):
        for g in range(CH // 16):
            dv[pl.ds(g * 16, 16)] = sd[1, pl.ds(g * 16, 16)]

    def gather(j, sd, buf, gsem):
        pltpu.async_copy(h_hbm.at[sd.at[0]], buf, gsem)

    def gwait(j, sd, buf, gsem):
        pltpu.make_async_copy(h_hbm.at[sd.at[0]], buf, gsem).wait()

    def scat(buf, dv):
        pltpu.sync_copy(buf, acc.at[dv], add=True)

    iload(0, sd0, isem0)
    iwait(0, sd0, isem0)
    dcopy(sd0, dv0)
    gather(0, sd0, rows0, gsem0)
    iload(1, sd1, isem1)

    def pair(k, _):
        i = 2 * k + 1
        iwait(i, sd1, isem1)
        dcopy(sd1, dv1)
        gather(i, sd1, rows1, gsem1)
        gwait(i - 1, sd0, rows0, gsem0)
        scat(rows0, dv0)
        iload(i + 1, sd0, isem0)
        iwait(i + 1, sd0, isem0)
        dcopy(sd0, dv0)
        gather(i + 1, sd0, rows0, gsem0)
        gwait(i, sd1, rows1, gsem1)
        scat(rows1, dv1)

        @pl.when(i + 2 < NCHUNK)
        def _():
            iload(i + 2, sd1, isem1)

        return 0

    lax.fori_loop(0, (NCHUNK - 1) // 2, pair, 0)
    gwait(NCHUNK - 1, sd0, rows0, gsem0)
    scat(rows0, dv0)
    plsc.subcore_barrier()

    for t in range(NRB):
        r0 = s * RPT + t * RB
        pltpu.sync_copy(acc.at[pl.ds(r0, RB)], rows0)
        pltpu.sync_copy(rows0, out_hbm.at[c, pl.ds(r0, RB)])


RBLK = 2000
GRID = N // RBLK


def _mm_scale_body(x_ref, w_ref, dega_ref, o_ref, dis_ref):
    deg = dega_ref[0, :, 0:1] + dega_ref[1, :, 0:1] + 1.0
    dis = lax.rsqrt(deg)
    dis_ref[...] = jnp.broadcast_to(dis, (RBLK, 16))
    h = jnp.dot(x_ref[...], w_ref[...], preferred_element_type=jnp.float32,
                precision=lax.Precision.HIGHEST)
    o_ref[...] = h * dis


def _mid_body(a_ref, hp_ref, dis_ref, b_ref, w_ref, o_ref):
    dis = dis_ref[:, 0:1]
    z = (a_ref[0] + a_ref[1] + hp_ref[...]) * dis + b_ref[...]
    z = jnp.maximum(z, 0.0)
    h = jnp.dot(z, w_ref[...], preferred_element_type=jnp.float32,
                precision=lax.Precision.HIGHEST)
    o_ref[...] = h * dis


def _final_body(a_ref, hp_ref, dis_ref, b_ref, o_ref):
    dis = dis_ref[:, 0:1]
    o_ref[...] = (a_ref[0] + a_ref[1] + hp_ref[...]) * dis + b_ref[...]


_row_spec = pl.BlockSpec((RBLK, D), lambda i: (i, 0))
_agg_spec = pl.BlockSpec((NC, RBLK, D), lambda i: (0, i, 0))
_dis_spec = pl.BlockSpec((RBLK, 16), lambda i: (i, 0))
_w_spec = pl.BlockSpec((D, D), lambda i: (0, 0))
_b_spec = pl.BlockSpec((1, D), lambda i: (0, 0))
_out_shape = jax.ShapeDtypeStruct((N, D), jnp.float32)

_mm_scale = pl.pallas_call(
    _mm_scale_body,
    grid=(GRID,),
    in_specs=[_row_spec, _w_spec, _agg_spec],
    out_specs=(_row_spec, _dis_spec),
    out_shape=(_out_shape, jax.ShapeDtypeStruct((N, 16), jnp.float32)),
)

_mid = pl.pallas_call(
    _mid_body,
    grid=(GRID,),
    in_specs=[_agg_spec, _row_spec, _dis_spec, _b_spec, _w_spec],
    out_specs=_row_spec,
    out_shape=_out_shape,
)

_final = pl.pallas_call(
    _final_body,
    grid=(GRID,),
    in_specs=[_agg_spec, _row_spec, _dis_spec, _b_spec],
    out_specs=_row_spec,
    out_shape=_out_shape,
)


def kernel(x, edge_index, W1, b1, W2, b2):
    src = edge_index[0].astype(jnp.int32)
    dst = edge_index[1].astype(jnp.int32)
    b1r = b1.reshape(1, D)
    b2r = b2.reshape(1, D)

    padlen = EPT_P - EPT
    w_ids = jnp.arange(NW, dtype=jnp.int32)[:, None]
    p_ids = jnp.arange(padlen, dtype=jnp.int32)[None, :]
    pad_src = (w_ids * 131 + p_ids * 997) % N
    pad_dst = N + (w_ids * 7 + p_ids) % (NP - N)
    srcp = jnp.concatenate([src.reshape(NW, EPT), pad_src], axis=1)
    dstp = jnp.concatenate([dst.reshape(NW, EPT), pad_dst], axis=1)
    sd = jnp.stack(
        [srcp.reshape(NW, NCHUNK, CH), dstp.reshape(NW, NCHUNK, CH)], axis=2
    )

    ones_t = jnp.ones((N, D), jnp.float32)
    dega = _agg_kernel(ones_t, sd)
    h1p, dis16 = _mm_scale(x, W1, dega)
    a1 = _agg_kernel(h1p, sd)
    h2p = _mid(a1, h1p, dis16, b1r, W2)
    a2 = _agg_kernel(h2p, sd)
    return _final(a2, h2p, dis16, b2r)


# speedup vs baseline: 22.8650x; 1.0132x over previous
import functools

import jax
import jax.numpy as jnp
from jax import lax
from jax.experimental import pallas as pl
from jax.experimental.pallas import tpu as pltpu
from jax.experimental.pallas import tpu_sc as plsc

N = 10000
NP = 10240
E = 320000
D = 128

NC = 2
NS = 16
NW = NC * NS

EPT = E // NW
CH = 128
NCHUNK = 81
EPT_P = NCHUNK * CH
RPT = NP // NS
RB = CH
NRB = RPT // RB

_mesh = plsc.VectorSubcoreMesh(
    core_axis_name="c", subcore_axis_name="s", num_cores=NC, num_subcores=NS
)


@functools.partial(
    pl.kernel,
    out_type=jax.ShapeDtypeStruct((NC, NP, D), jnp.float32),
    mesh=_mesh,
    scratch_types=(
        pltpu.VMEM_SHARED((NP, D), jnp.float32),
        pltpu.VMEM((2, CH), jnp.int32),
        pltpu.VMEM((2, CH), jnp.int32),
        pltpu.VMEM((CH,), jnp.int32),
        pltpu.VMEM((CH,), jnp.int32),
        pltpu.VMEM((CH, D), jnp.float32),
        pltpu.VMEM((CH, D), jnp.float32),
        pltpu.SemaphoreType.DMA,
        pltpu.SemaphoreType.DMA,
        pltpu.SemaphoreType.DMA,
        pltpu.SemaphoreType.DMA,
    ),
)
def _agg_kernel(h_hbm, sd_hbm, out_hbm,
                acc, sd0, sd1, dv0, dv1, rows0, rows1,
                gsem0, gsem1, isem0, isem1):
    c = lax.axis_index("c")
    s = lax.axis_index("s")
    wid = c * NS + s

    def fillz(i, _):
        for j in range(D // 16):
            rows0[i, pl.ds(j * 16, 16)] = jnp.zeros((16,), jnp.float32)
        return 0

    lax.fori_loop(0, RB, fillz, 0)
    for t in range(NRB):
        pltpu.async_copy(rows0, acc.at[pl.ds(s * RPT + t * RB, RB)], gsem0)
    for t in range(NRB):
        pltpu.make_async_copy(
            rows0, acc.at[pl.ds(s * RPT + t * RB, RB)], gsem0).wait()
    plsc.subcore_barrier()

    def iload(j, sd, isem):
        pltpu.async_copy(sd_hbm.at[wid, j], sd, isem)

    def iwait(j, sd, isem):
        pltpu.make_async_copy(sd_hbm.at[wid, j], sd, isem).wait()

    def dcopy(sd, dv):
        for g in range(CH // 16):
            dv[pl.ds(g * 16, 16)] = sd[1, pl.ds(g * 16, 16)]

    def gather(j, sd, buf, gsem):
        pltpu.async_copy(h_hbm.at[sd.at[0]], buf, gsem)

    def gwait(j, sd, buf, gsem):
        pltpu.make_async_copy(h_hbm.at[sd.at[0]], buf, gsem).wait()

    def scat(buf, dv):
        pltpu.sync_copy(buf, acc.at[dv], add=True)

    iload(0, sd0, isem0)
    iwait(0, sd0, isem0)
    dcopy(sd0, dv0)
    gather(0, sd0, rows0, gsem0)
    iload(1, sd1, isem1)

    def pair(k, _):
        i = 2 * k + 1
        iwait(i, sd1, isem1)
        dcopy(sd1, dv1)
        gather(i, sd1, rows1, gsem1)
        gwait(i - 1, sd0, rows0, gsem0)
        scat(rows0, dv0)
        iload(i + 1, sd0, isem0)
        iwait(i + 1, sd0, isem0)
        dcopy(sd0, dv0)
        gather(i + 1, sd0, rows0, gsem0)
        gwait(i, sd1, rows1, gsem1)
        scat(rows1, dv1)

        @pl.when(i + 2 < NCHUNK)
        def _():
            iload(i + 2, sd1, isem1)

        return 0

    lax.fori_loop(0, (NCHUNK - 1) // 2, pair, 0)
    gwait(NCHUNK - 1, sd0, rows0, gsem0)
    scat(rows0, dv0)
    plsc.subcore_barrier()

    def rb_slice(t):
        return pl.ds(s * RPT + t * RB, RB)

    bufs = (rows0, rows1)
    sems = (gsem0, gsem1)
    pltpu.async_copy(acc.at[rb_slice(0)], rows0, gsem0)
    for t in range(NRB):
        if t >= 1:
            pltpu.make_async_copy(bufs[(t - 1) % 2],
                                  out_hbm.at[c, rb_slice(t - 1)], isem0).wait()
        if t + 1 < NRB:
            pltpu.async_copy(acc.at[rb_slice(t + 1)], bufs[(t + 1) % 2],
                             sems[(t + 1) % 2])
        pltpu.make_async_copy(acc.at[rb_slice(t)], bufs[t % 2],
                              sems[t % 2]).wait()
        pltpu.async_copy(bufs[t % 2], out_hbm.at[c, rb_slice(t)], isem0)
    pltpu.make_async_copy(bufs[(NRB - 1) % 2],
                          out_hbm.at[c, rb_slice(NRB - 1)], isem0).wait()


RBLK = 2000
GRID = N // RBLK


def _mm_scale_body(x_ref, w_ref, dega_ref, o_ref, dis_ref):
    deg = dega_ref[0, :, 0:1] + dega_ref[1, :, 0:1] + 1.0
    dis = lax.rsqrt(deg)
    dis_ref[...] = jnp.broadcast_to(dis, (RBLK, 16))
    h = jnp.dot(x_ref[...], w_ref[...], preferred_element_type=jnp.float32,
                precision=lax.Precision.HIGHEST)
    o_ref[...] = h * dis


def _mid_body(a_ref, hp_ref, dis_ref, b_ref, w_ref, o_ref):
    dis = dis_ref[:, 0:1]
    z = (a_ref[0] + a_ref[1] + hp_ref[...]) * dis + b_ref[...]
    z = jnp.maximum(z, 0.0)
    h = jnp.dot(z, w_ref[...], preferred_element_type=jnp.float32,
                precision=lax.Precision.HIGHEST)
    o_ref[...] = h * dis


def _final_body(a_ref, hp_ref, dis_ref, b_ref, o_ref):
    dis = dis_ref[:, 0:1]
    o_ref[...] = (a_ref[0] + a_ref[1] + hp_ref[...]) * dis + b_ref[...]


_row_spec = pl.BlockSpec((RBLK, D), lambda i: (i, 0))
_agg_spec = pl.BlockSpec((NC, RBLK, D), lambda i: (0, i, 0))
_dis_spec = pl.BlockSpec((RBLK, 16), lambda i: (i, 0))
_w_spec = pl.BlockSpec((D, D), lambda i: (0, 0))
_b_spec = pl.BlockSpec((1, D), lambda i: (0, 0))
_out_shape = jax.ShapeDtypeStruct((N, D), jnp.float32)

_mm_scale = pl.pallas_call(
    _mm_scale_body,
    grid=(GRID,),
    in_specs=[_row_spec, _w_spec, _agg_spec],
    out_specs=(_row_spec, _dis_spec),
    out_shape=(_out_shape, jax.ShapeDtypeStruct((N, 16), jnp.float32)),
)

_mid = pl.pallas_call(
    _mid_body,
    grid=(GRID,),
    in_specs=[_agg_spec, _row_spec, _dis_spec, _b_spec, _w_spec],
    out_specs=_row_spec,
    out_shape=_out_shape,
)

_final = pl.pallas_call(
    _final_body,
    grid=(GRID,),
    in_specs=[_agg_spec, _row_spec, _dis_spec, _b_spec],
    out_specs=_row_spec,
    out_shape=_out_shape,
)


def kernel(x, edge_index, W1, b1, W2, b2):
    src = edge_index[0].astype(jnp.int32)
    dst = edge_index[1].astype(jnp.int32)
    b1r = b1.reshape(1, D)
    b2r = b2.reshape(1, D)

    padlen = EPT_P - EPT
    w_ids = jnp.arange(NW, dtype=jnp.int32)[:, None]
    p_ids = jnp.arange(padlen, dtype=jnp.int32)[None, :]
    pad_src = (w_ids * 131 + p_ids * 997) % N
    pad_dst = N + (w_ids * 7 + p_ids) % (NP - N)
    srcp = jnp.concatenate([src.reshape(NW, EPT), pad_src], axis=1)
    dstp = jnp.concatenate([dst.reshape(NW, EPT), pad_dst], axis=1)
    sd = jnp.stack(
        [srcp.reshape(NW, NCHUNK, CH), dstp.reshape(NW, NCHUNK, CH)], axis=2
    )

    ones_t = jnp.ones((N, D), jnp.float32)
    dega = _agg_kernel(ones_t, sd)
    h1p, dis16 = _mm_scale(x, W1, dega)
    a1 = _agg_kernel(h1p, sd)
    h2p = _mid(a1, h1p, dis16, b1r, W2)
    a2 = _agg_kernel(h2p, sd)
    return _final(a2, h2p, dis16, b2r)


# speedup vs baseline: 23.0196x; 1.0068x over previous
import functools

import jax
import jax.numpy as jnp
from jax import lax
from jax.experimental import pallas as pl
from jax.experimental.pallas import tpu as pltpu
from jax.experimental.pallas import tpu_sc as plsc

N = 10000
NP = 10240
E = 320000
D = 128

NC = 2
NS = 16
NW = NC * NS

EPT = E // NW
CH = 128
NCHUNK = 81
EPT_P = NCHUNK * CH
RPT = NP // NS
RB = CH
NRB = RPT // RB

_mesh = plsc.VectorSubcoreMesh(
    core_axis_name="c", subcore_axis_name="s", num_cores=NC, num_subcores=NS
)


@functools.partial(
    pl.kernel,
    out_type=jax.ShapeDtypeStruct((NC, NP, D), jnp.float32),
    mesh=_mesh,
    scratch_types=(
        pltpu.VMEM_SHARED((NP, D), jnp.float32),
        pltpu.VMEM((2, CH), jnp.int32),
        pltpu.VMEM((2, CH), jnp.int32),
        pltpu.VMEM((CH,), jnp.int32),
        pltpu.VMEM((CH,), jnp.int32),
        pltpu.VMEM((CH, D), jnp.float32),
        pltpu.VMEM((CH, D), jnp.float32),
        pltpu.SemaphoreType.DMA,
        pltpu.SemaphoreType.DMA,
        pltpu.SemaphoreType.DMA,
        pltpu.SemaphoreType.DMA,
    ),
)
def _agg_kernel(h_hbm, sd_hbm, out_hbm,
                acc, sd0, sd1, dv0, dv1, rows0, rows1,
                gsem0, gsem1, isem0, isem1):
    c = lax.axis_index("c")
    s = lax.axis_index("s")
    wid = c * NS + s

    def fillz(i, _):
        for j in range(D // 16):
            rows0[i, pl.ds(j * 16, 16)] = jnp.zeros((16,), jnp.float32)
        return 0

    lax.fori_loop(0, RB, fillz, 0)
    for t in range(NRB):
        pltpu.async_copy(rows0, acc.at[pl.ds(s * RPT + t * RB, RB)], gsem0)
    for t in range(NRB):
        pltpu.make_async_copy(
            rows0, acc.at[pl.ds(s * RPT + t * RB, RB)], gsem0).wait()
    plsc.subcore_barrier()

    def iload(j, sd, isem):
        pltpu.async_copy(sd_hbm.at[wid, j], sd, isem)

    def iwait(j, sd, isem):
        pltpu.make_async_copy(sd_hbm.at[wid, j], sd, isem).wait()

    def dcopy(sd, dv):
        for g in range(CH // 16):
            dv[pl.ds(g * 16, 16)] = sd[1, pl.ds(g * 16, 16)]

    def gather(j, sd, buf, gsem):
        pltpu.async_copy(h_hbm.at[sd.at[0]], buf, gsem)

    def gwait(j, sd, buf, gsem):
        pltpu.make_async_copy(h_hbm.at[sd.at[0]], buf, gsem).wait()

    def scat(buf, dv):
        pltpu.sync_copy(buf, acc.at[dv], add=True)

    iload(0, sd0, isem0)
    iwait(0, sd0, isem0)
    dcopy(sd0, dv0)
    gather(0, sd0, rows0, gsem0)
    iload(1, sd1, isem1)

    def pair(k, _):
        i = 2 * k + 1
        iwait(i, sd1, isem1)
        dcopy(sd1, dv1)
        gather(i, sd1, rows1, gsem1)
        gwait(i - 1, sd0, rows0, gsem0)
        scat(rows0, dv0)
        iload(i + 1, sd0, isem0)
        iwait(i + 1, sd0, isem0)
        dcopy(sd0, dv0)
        gather(i + 1, sd0, rows0, gsem0)
        gwait(i, sd1, rows1, gsem1)
        scat(rows1, dv1)

        @pl.when(i + 2 < NCHUNK)
        def _():
            iload(i + 2, sd1, isem1)

        return 0

    lax.fori_loop(0, (NCHUNK - 1) // 2, pair, 0)
    gwait(NCHUNK - 1, sd0, rows0, gsem0)
    scat(rows0, dv0)
    plsc.subcore_barrier()

    def rb_slice(t):
        return pl.ds(s * RPT + t * RB, RB)

    bufs = (rows0, rows1)
    sems = (gsem0, gsem1)
    pltpu.async_copy(acc.at[rb_slice(0)], rows0, gsem0)
    for t in range(NRB):
        if t >= 1:
            pltpu.make_async_copy(bufs[(t - 1) % 2],
                                  out_hbm.at[c, rb_slice(t - 1)], isem0).wait()
        if t + 1 < NRB:
            pltpu.async_copy(acc.at[rb_slice(t + 1)], bufs[(t + 1) % 2],
                             sems[(t + 1) % 2])
        pltpu.make_async_copy(acc.at[rb_slice(t)], bufs[t % 2],
                              sems[t % 2]).wait()
        pltpu.async_copy(bufs[t % 2], out_hbm.at[c, rb_slice(t)], isem0)
    pltpu.make_async_copy(bufs[(NRB - 1) % 2],
                          out_hbm.at[c, rb_slice(NRB - 1)], isem0).wait()


RBLK = 2000
GRID = N // RBLK


def _mm_scale_body(x_ref, w_ref, dega_ref, o_ref, dis_ref):
    deg = dega_ref[0, :, 0:1] + dega_ref[1, :, 0:1] + 1.0
    dis = lax.rsqrt(deg)
    dis_ref[...] = jnp.broadcast_to(dis, (RBLK, 16))
    h = jnp.dot(x_ref[...], w_ref[...], preferred_element_type=jnp.float32)
    o_ref[...] = h * dis


def _mid_body(a_ref, hp_ref, dis_ref, b_ref, w_ref, o_ref):
    dis = dis_ref[:, 0:1]
    z = (a_ref[0] + a_ref[1] + hp_ref[...]) * dis + b_ref[...]
    z = jnp.maximum(z, 0.0)
    h = jnp.dot(z, w_ref[...], preferred_element_type=jnp.float32)
    o_ref[...] = h * dis


def _final_body(a_ref, hp_ref, dis_ref, b_ref, o_ref):
    dis = dis_ref[:, 0:1]
    o_ref[...] = (a_ref[0] + a_ref[1] + hp_ref[...]) * dis + b_ref[...]


_row_spec = pl.BlockSpec((RBLK, D), lambda i: (i, 0))
_agg_spec = pl.BlockSpec((NC, RBLK, D), lambda i: (0, i, 0))
_dis_spec = pl.BlockSpec((RBLK, 16), lambda i: (i, 0))
_w_spec = pl.BlockSpec((D, D), lambda i: (0, 0))
_b_spec = pl.BlockSpec((1, D), lambda i: (0, 0))
_out_shape = jax.ShapeDtypeStruct((N, D), jnp.float32)

_mm_scale = pl.pallas_call(
    _mm_scale_body,
    grid=(GRID,),
    in_specs=[_row_spec, _w_spec, _agg_spec],
    out_specs=(_row_spec, _dis_spec),
    out_shape=(_out_shape, jax.ShapeDtypeStruct((N, 16), jnp.float32)),
)

_mid = pl.pallas_call(
    _mid_body,
    grid=(GRID,),
    in_specs=[_agg_spec, _row_spec, _dis_spec, _b_spec, _w_spec],
    out_specs=_row_spec,
    out_shape=_out_shape,
)

_final = pl.pallas_call(
    _final_body,
    grid=(GRID,),
    in_specs=[_agg_spec, _row_spec, _dis_spec, _b_spec],
    out_specs=_row_spec,
    out_shape=_out_shape,
)


def kernel(x, edge_index, W1, b1, W2, b2):
    src = edge_index[0].astype(jnp.int32)
    dst = edge_index[1].astype(jnp.int32)
    b1r = b1.reshape(1, D)
    b2r = b2.reshape(1, D)

    padlen = EPT_P - EPT
    w_ids = jnp.arange(NW, dtype=jnp.int32)[:, None]
    p_ids = jnp.arange(padlen, dtype=jnp.int32)[None, :]
    pad_src = (w_ids * 131 + p_ids * 997) % N
    pad_dst = N + (w_ids * 7 + p_ids) % (NP - N)
    srcp = jnp.concatenate([src.reshape(NW, EPT), pad_src], axis=1)
    dstp = jnp.concatenate([dst.reshape(NW, EPT), pad_dst], axis=1)
    sd = jnp.stack(
        [srcp.reshape(NW, NCHUNK, CH), dstp.reshape(NW, NCHUNK, CH)], axis=2
    )

    ones_t = jnp.ones((N, D), jnp.float32)
    dega = _agg_kernel(ones_t, sd)
    h1p, dis16 = _mm_scale(x, W1, dega)
    a1 = _agg_kernel(h1p, sd)
    h2p = _mid(a1, h1p, dis16, b1r, W2)
    a2 = _agg_kernel(h2p, sd)
    return _final(a2, h2p, dis16, b2r)


# speedup vs baseline: 23.2353x; 1.0094x over previous
import functools

import jax
import jax.numpy as jnp
from jax import lax
from jax.experimental import pallas as pl
from jax.experimental.pallas import tpu as pltpu
from jax.experimental.pallas import tpu_sc as plsc

N = 10000
NP = 10240
E = 320000
D = 128

NC = 2
NS = 16
NW = NC * NS

EPT = E // NW
CH = 128
NCHUNK = 81
EPT_P = NCHUNK * CH
RPT = NP // NS
RB = CH
NRB = RPT // RB

_mesh = plsc.VectorSubcoreMesh(
    core_axis_name="c", subcore_axis_name="s", num_cores=NC, num_subcores=NS
)


@functools.partial(
    pl.kernel,
    out_type=jax.ShapeDtypeStruct((NC, NP, D), jnp.float32),
    mesh=_mesh,
    scratch_types=(
        pltpu.VMEM_SHARED((NP, D), jnp.float32),
        pltpu.VMEM((2, CH), jnp.int32),
        pltpu.VMEM((2, CH), jnp.int32),
        pltpu.VMEM((CH,), jnp.int32),
        pltpu.VMEM((CH,), jnp.int32),
        pltpu.VMEM((CH, D), jnp.float32),
        pltpu.VMEM((CH, D), jnp.float32),
        pltpu.SemaphoreType.DMA,
        pltpu.SemaphoreType.DMA,
        pltpu.SemaphoreType.DMA,
        pltpu.SemaphoreType.DMA,
    ),
)
def _agg_kernel(h_hbm, sd_hbm, out_hbm,
                acc, sd0, sd1, dv0, dv1, rows0, rows1,
                gsem0, gsem1, isem0, isem1):
    c = lax.axis_index("c")
    s = lax.axis_index("s")
    wid = c * NS + s

    def fillz(i, _):
        for j in range(D // 16):
            rows1[i, pl.ds(j * 16, 16)] = jnp.zeros((16,), jnp.float32)
        return 0

    lax.fori_loop(0, RB, fillz, 0)

    def iload(j, sd, isem):
        pltpu.async_copy(sd_hbm.at[wid, j], sd, isem)

    def iwait(j, sd, isem):
        pltpu.make_async_copy(sd_hbm.at[wid, j], sd, isem).wait()

    def dcopy(sd, dv):
        for g in range(CH // 16):
            dv[pl.ds(g * 16, 16)] = sd[1, pl.ds(g * 16, 16)]

    def gather(j, sd, buf, gsem):
        pltpu.async_copy(h_hbm.at[sd.at[0]], buf, gsem)

    def gwait(j, sd, buf, gsem):
        pltpu.make_async_copy(h_hbm.at[sd.at[0]], buf, gsem).wait()

    def scat(buf, dv):
        pltpu.sync_copy(buf, acc.at[dv], add=True)

    iload(0, sd0, isem0)
    for t in range(NRB):
        pltpu.async_copy(rows1, acc.at[pl.ds(s * RPT + t * RB, RB)], gsem1)
    iwait(0, sd0, isem0)
    dcopy(sd0, dv0)
    gather(0, sd0, rows0, gsem0)
    iload(1, sd1, isem1)
    for t in range(NRB):
        pltpu.make_async_copy(
            rows1, acc.at[pl.ds(s * RPT + t * RB, RB)], gsem1).wait()
    plsc.subcore_barrier()

    def pair(k, _):
        i = 2 * k + 1
        iwait(i, sd1, isem1)
        dcopy(sd1, dv1)
        gather(i, sd1, rows1, gsem1)
        gwait(i - 1, sd0, rows0, gsem0)
        scat(rows0, dv0)
        iload(i + 1, sd0, isem0)
        iwait(i + 1, sd0, isem0)
        dcopy(sd0, dv0)
        gather(i + 1, sd0, rows0, gsem0)
        gwait(i, sd1, rows1, gsem1)
        scat(rows1, dv1)

        @pl.when(i + 2 < NCHUNK)
        def _():
            iload(i + 2, sd1, isem1)

        return 0

    lax.fori_loop(0, (NCHUNK - 1) // 2, pair, 0)
    gwait(NCHUNK - 1, sd0, rows0, gsem0)
    scat(rows0, dv0)
    plsc.subcore_barrier()

    def rb_slice(t):
        return pl.ds(s * RPT + t * RB, RB)

    bufs = (rows0, rows1)
    sems = (gsem0, gsem1)
    pltpu.async_copy(acc.at[rb_slice(0)], rows0, gsem0)
    for t in range(NRB):
        if t >= 1:
            pltpu.make_async_copy(bufs[(t - 1) % 2],
                                  out_hbm.at[c, rb_slice(t - 1)], isem0).wait()
        if t + 1 < NRB:
            pltpu.async_copy(acc.at[rb_slice(t + 1)], bufs[(t + 1) % 2],
                             sems[(t + 1) % 2])
        pltpu.make_async_copy(acc.at[rb_slice(t)], bufs[t % 2],
                              sems[t % 2]).wait()
        pltpu.async_copy(bufs[t % 2], out_hbm.at[c, rb_slice(t)], isem0)
    pltpu.make_async_copy(bufs[(NRB - 1) % 2],
                          out_hbm.at[c, rb_slice(NRB - 1)], isem0).wait()


RBLK = 2000
GRID = N // RBLK


def _mm_scale_body(x_ref, w_ref, dega_ref, o_ref, dis_ref):
    deg = dega_ref[0, :, 0:1] + dega_ref[1, :, 0:1] + 1.0
    dis = lax.rsqrt(deg)
    dis_ref[...] = jnp.broadcast_to(dis, (RBLK, 16))
    h = jnp.dot(x_ref[...], w_ref[...], preferred_element_type=jnp.float32)
    o_ref[...] = h * dis


def _mid_body(a_ref, hp_ref, dis_ref, b_ref, w_ref, o_ref):
    dis = dis_ref[:, 0:1]
    z = (a_ref[0] + a_ref[1] + hp_ref[...]) * dis + b_ref[...]
    z = jnp.maximum(z, 0.0)
    h = jnp.dot(z, w_ref[...], preferred_element_type=jnp.float32)
    o_ref[...] = h * dis


def _final_body(a_ref, hp_ref, dis_ref, b_ref, o_ref):
    dis = dis_ref[:, 0:1]
    o_ref[...] = (a_ref[0] + a_ref[1] + hp_ref[...]) * dis + b_ref[...]


_row_spec = pl.BlockSpec((RBLK, D), lambda i: (i, 0))
_agg_spec = pl.BlockSpec((NC, RBLK, D), lambda i: (0, i, 0))
_dis_spec = pl.BlockSpec((RBLK, 16), lambda i: (i, 0))
_w_spec = pl.BlockSpec((D, D), lambda i: (0, 0))
_b_spec = pl.BlockSpec((1, D), lambda i: (0, 0))
_out_shape = jax.ShapeDtypeStruct((N, D), jnp.float32)

_mm_scale = pl.pallas_call(
    _mm_scale_body,
    grid=(GRID,),
    in_specs=[_row_spec, _w_spec, _agg_spec],
    out_specs=(_row_spec, _dis_spec),
    out_shape=(_out_shape, jax.ShapeDtypeStruct((N, 16), jnp.float32)),
)

_mid = pl.pallas_call(
    _mid_body,
    grid=(GRID,),
    in_specs=[_agg_spec, _row_spec, _dis_spec, _b_spec, _w_spec],
    out_specs=_row_spec,
    out_shape=_out_shape,
)

_final = pl.pallas_call(
    _final_body,
    grid=(GRID,),
    in_specs=[_agg_spec, _row_spec, _dis_spec, _b_spec],
    out_specs=_row_spec,
    out_shape=_out_shape,
)


def kernel(x, edge_index, W1, b1, W2, b2):
    src = edge_index[0].astype(jnp.int32)
    dst = edge_index[1].astype(jnp.int32)
    b1r = b1.reshape(1, D)
    b2r = b2.reshape(1, D)

    padlen = EPT_P - EPT
    w_ids = jnp.arange(NW, dtype=jnp.int32)[:, None]
    p_ids = jnp.arange(padlen, dtype=jnp.int32)[None, :]
    pad_src = (w_ids * 131 + p_ids * 997) % N
    pad_dst = N + (w_ids * 7 + p_ids) % (NP - N)
    srcp = jnp.concatenate([src.reshape(NW, EPT), pad_src], axis=1)
    dstp = jnp.concatenate([dst.reshape(NW, EPT), pad_dst], axis=1)
    sd = jnp.stack(
        [srcp.reshape(NW, NCHUNK, CH), dstp.reshape(NW, NCHUNK, CH)], axis=2
    )

    ones_t = jnp.ones((N, D), jnp.float32)
    dega = _agg_kernel(ones_t, sd)
    h1p, dis16 = _mm_scale(x, W1, dega)
    a1 = _agg_kernel(h1p, sd)
    h2p = _mid(a1, h1p, dis16, b1r, W2)
    a2 = _agg_kernel(h2p, sd)
    return _final(a2, h2p, dis16, b2r)


# speedup vs baseline: 26.1021x; 1.1234x over previous
import functools

import jax
import jax.numpy as jnp
from jax import lax
from jax.experimental import pallas as pl
from jax.experimental.pallas import tpu as pltpu
from jax.experimental.pallas import tpu_sc as plsc

N = 10000
NP = 10240
E = 320000
D = 128

NC = 2
NS = 16
NW = NC * NS

EPT = E // NW
CH = 128
NCHUNK = 81
EPT_P = NCHUNK * CH
RPT = NP // NS
RB = CH
NRB = RPT // RB

_mesh = plsc.VectorSubcoreMesh(
    core_axis_name="c", subcore_axis_name="s", num_cores=NC, num_subcores=NS
)


@functools.partial(
    pl.kernel,
    out_type=jax.ShapeDtypeStruct((NC, NP, D), jnp.float32),
    mesh=_mesh,
    scratch_types=(
        pltpu.VMEM_SHARED((NP, D), jnp.float32),
        pltpu.VMEM((2, CH), jnp.int32),
        pltpu.VMEM((2, CH), jnp.int32),
        pltpu.VMEM((CH,), jnp.int32),
        pltpu.VMEM((CH,), jnp.int32),
        pltpu.VMEM((CH,), jnp.int32),
        pltpu.VMEM((CH,), jnp.int32),
        pltpu.VMEM((CH, D), jnp.float32),
        pltpu.VMEM((CH, D), jnp.float32),
        pltpu.SemaphoreType.DMA,
        pltpu.SemaphoreType.DMA,
        pltpu.SemaphoreType.DMA,
        pltpu.SemaphoreType.DMA,
        pltpu.SemaphoreType.DMA,
        pltpu.SemaphoreType.DMA,
    ),
)
def _agg_kernel(h_hbm, sd_hbm, out_hbm,
                acc, sd0, sd1, dv0, dv1, dv2, dv3, rows0, rows1,
                gsem0, gsem1, isem0, isem1, ssem0, ssem1):
    c = lax.axis_index("c")
    s = lax.axis_index("s")
    wid = c * NS + s

    def fillz(i, _):
        for j in range(D // 16):
            rows1[i, pl.ds(j * 16, 16)] = jnp.zeros((16,), jnp.float32)
        return 0

    lax.fori_loop(0, RB, fillz, 0)

    def iload(j, sd, isem):
        pltpu.async_copy(sd_hbm.at[wid, j], sd, isem)

    def iwait(j, sd, isem):
        pltpu.make_async_copy(sd_hbm.at[wid, j], sd, isem).wait()

    def dcopy(sd, dv):
        for g in range(CH // 16):
            dv[pl.ds(g * 16, 16)] = sd[1, pl.ds(g * 16, 16)]

    def gather(j, sd, buf, gsem):
        pltpu.async_copy(h_hbm.at[sd.at[0]], buf, gsem)

    def gwait(j, sd, buf, gsem):
        pltpu.make_async_copy(h_hbm.at[sd.at[0]], buf, gsem).wait()

    def sstart(buf, dv, ssem):
        pltpu.async_copy(buf, acc.at[dv], ssem, add=True)

    def swait(buf, dv, ssem):
        pltpu.make_async_copy(buf, acc.at[dv], ssem).wait()


    iload(0, sd0, isem0)
    for t in range(NRB):
        pltpu.async_copy(rows1, acc.at[pl.ds(s * RPT + t * RB, RB)], gsem1)
    iwait(0, sd0, isem0)
    dcopy(sd0, dv0)
    gather(0, sd0, rows0, gsem0)
    iload(1, sd1, isem1)
    for t in range(NRB):
        pltpu.make_async_copy(
            rows1, acc.at[pl.ds(s * RPT + t * RB, RB)], gsem1).wait()
    plsc.subcore_barrier()

    iwait(1, sd1, isem1)
    dcopy(sd1, dv1)
    gather(1, sd1, rows1, gsem1)
    gwait(0, sd0, rows0, gsem0)
    sstart(rows0, dv0, ssem0)
    iload(2, sd0, isem0)
    iwait(2, sd0, isem0)
    dcopy(sd0, dv2)
    swait(rows0, dv0, ssem0)
    gather(2, sd0, rows0, gsem0)
    gwait(1, sd1, rows1, gsem1)
    sstart(rows1, dv1, ssem1)
    iload(3, sd1, isem1)

    def quad(m, _):
        j0 = 4 * m + 3

        iwait(j0, sd1, isem1)
        dcopy(sd1, dv3)
        swait(rows1, dv1, ssem1)
        gather(j0, sd1, rows1, gsem1)
        gwait(j0 - 1, sd0, rows0, gsem0)
        sstart(rows0, dv2, ssem0)
        iload(j0 + 1, sd0, isem0)

        iwait(j0 + 1, sd0, isem0)
        dcopy(sd0, dv0)
        swait(rows0, dv2, ssem0)
        gather(j0 + 1, sd0, rows0, gsem0)
        gwait(j0, sd1, rows1, gsem1)
        sstart(rows1, dv3, ssem1)
        iload(j0 + 2, sd1, isem1)

        iwait(j0 + 2, sd1, isem1)
        dcopy(sd1, dv1)
        swait(rows1, dv3, ssem1)
        gather(j0 + 2, sd1, rows1, gsem1)
        gwait(j0 + 1, sd0, rows0, gsem0)
        sstart(rows0, dv0, ssem0)
        iload(j0 + 3, sd0, isem0)

        iwait(j0 + 3, sd0, isem0)
        dcopy(sd0, dv2)
        swait(rows0, dv0, ssem0)
        gather(j0 + 3, sd0, rows0, gsem0)
        gwait(j0 + 2, sd1, rows1, gsem1)
        sstart(rows1, dv1, ssem1)
        iload(j0 + 4, sd1, isem1)
        return 0

    lax.fori_loop(0, (NCHUNK - 5) // 4, quad, 0)

    jo = NCHUNK - 2
    iwait(jo, sd1, isem1)
    dcopy(sd1, dv3)
    swait(rows1, dv1, ssem1)
    gather(jo, sd1, rows1, gsem1)
    gwait(jo - 1, sd0, rows0, gsem0)
    sstart(rows0, dv2, ssem0)
    iload(jo + 1, sd0, isem0)
    iwait(jo + 1, sd0, isem0)
    dcopy(sd0, dv0)
    swait(rows0, dv2, ssem0)
    gather(jo + 1, sd0, rows0, gsem0)
    gwait(jo, sd1, rows1, gsem1)
    sstart(rows1, dv3, ssem1)
    gwait(jo + 1, sd0, rows0, gsem0)
    sstart(rows0, dv0, ssem0)
    swait(rows1, dv3, ssem1)
    swait(rows0, dv0, ssem0)
    plsc.subcore_barrier()

    def rb_slice(t):
        return pl.ds(s * RPT + t * RB, RB)

    bufs = (rows0, rows1)
    sems = (gsem0, gsem1)
    pltpu.async_copy(acc.at[rb_slice(0)], rows0, gsem0)
    for t in range(NRB):
        if t >= 1:
            pltpu.make_async_copy(bufs[(t - 1) % 2],
                                  out_hbm.at[c, rb_slice(t - 1)], isem0).wait()
        if t + 1 < NRB:
            pltpu.async_copy(acc.at[rb_slice(t + 1)], bufs[(t + 1) % 2],
                             sems[(t + 1) % 2])
        pltpu.make_async_copy(acc.at[rb_slice(t)], bufs[t % 2],
                              sems[t % 2]).wait()
        pltpu.async_copy(bufs[t % 2], out_hbm.at[c, rb_slice(t)], isem0)
    pltpu.make_async_copy(bufs[(NRB - 1) % 2],
                          out_hbm.at[c, rb_slice(NRB - 1)], isem0).wait()


RBLK = 2000
GRID = N // RBLK


def _mm_scale_body(x_ref, w_ref, dega_ref, o_ref, dis_ref):
    deg = dega_ref[0, :, 0:1] + dega_ref[1, :, 0:1] + 1.0
    dis = lax.rsqrt(deg)
    dis_ref[...] = jnp.broadcast_to(dis, (RBLK, 16))
    h = jnp.dot(x_ref[...], w_ref[...], preferred_element_type=jnp.float32)
    o_ref[...] = h * dis


def _mid_body(a_ref, hp_ref, dis_ref, b_ref, w_ref, o_ref):
    dis = dis_ref[:, 0:1]
    z = (a_ref[0] + a_ref[1] + hp_ref[...]) * dis + b_ref[...]
    z = jnp.maximum(z, 0.0)
    h = jnp.dot(z, w_ref[...], preferred_element_type=jnp.float32)
    o_ref[...] = h * dis


def _final_body(a_ref, hp_ref, dis_ref, b_ref, o_ref):
    dis = dis_ref[:, 0:1]
    o_ref[...] = (a_ref[0] + a_ref[1] + hp_ref[...]) * dis + b_ref[...]


_row_spec = pl.BlockSpec((RBLK, D), lambda i: (i, 0))
_agg_spec = pl.BlockSpec((NC, RBLK, D), lambda i: (0, i, 0))
_dis_spec = pl.BlockSpec((RBLK, 16), lambda i: (i, 0))
_w_spec = pl.BlockSpec((D, D), lambda i: (0, 0))
_b_spec = pl.BlockSpec((1, D), lambda i: (0, 0))
_out_shape = jax.ShapeDtypeStruct((N, D), jnp.float32)

_mm_scale = pl.pallas_call(
    _mm_scale_body,
    grid=(GRID,),
    in_specs=[_row_spec, _w_spec, _agg_spec],
    out_specs=(_row_spec, _dis_spec),
    out_shape=(_out_shape, jax.ShapeDtypeStruct((N, 16), jnp.float32)),
)

_mid = pl.pallas_call(
    _mid_body,
    grid=(GRID,),
    in_specs=[_agg_spec, _row_spec, _dis_spec, _b_spec, _w_spec],
    out_specs=_row_spec,
    out_shape=_out_shape,
)

_final = pl.pallas_call(
    _final_body,
    grid=(GRID,),
    in_specs=[_agg_spec, _row_spec, _dis_spec, _b_spec],
    out_specs=_row_spec,
    out_shape=_out_shape,
)


def kernel(x, edge_index, W1, b1, W2, b2):
    src = edge_index[0].astype(jnp.int32)
    dst = edge_index[1].astype(jnp.int32)
    b1r = b1.reshape(1, D)
    b2r = b2.reshape(1, D)

    padlen = EPT_P - EPT
    w_ids = jnp.arange(NW, dtype=jnp.int32)[:, None]
    p_ids = jnp.arange(padlen, dtype=jnp.int32)[None, :]
    pad_src = (w_ids * 131 + p_ids * 997) % N
    pad_dst = N + (w_ids * 7 + p_ids) % (NP - N)
    srcp = jnp.concatenate([src.reshape(NW, EPT), pad_src], axis=1)
    dstp = jnp.concatenate([dst.reshape(NW, EPT), pad_dst], axis=1)
    sd = jnp.stack(
        [srcp.reshape(NW, NCHUNK, CH), dstp.reshape(NW, NCHUNK, CH)], axis=2
    )

    ones_t = jnp.ones((N, D), jnp.float32)
    dega = _agg_kernel(ones_t, sd)
    h1p, dis16 = _mm_scale(x, W1, dega)
    a1 = _agg_kernel(h1p, sd)
    h2p = _mid(a1, h1p, dis16, b1r, W2)
    a2 = _agg_kernel(h2p, sd)
    return _final(a2, h2p, dis16, b2r)
